# Initial kernel scaffold; baseline (speedup 1.0000x reference)
import jax
import jax.numpy as jnp
from jax.experimental import pallas as pl


def kernel(quantized):
    raise NotImplementedError("write your pallas kernel here")


# speedup vs baseline: 3.6479x; 3.6479x over previous
import jax
import jax.numpy as jnp
from jax.experimental import pallas as pl

_PROB = 0.12


def _neighbor_indices(T):
    k1, k2 = jax.random.split(jax.random.key(42))
    replace = jax.random.bernoulli(k1, _PROB, (T,))
    direction = jnp.where(jax.random.bernoulli(k2, 0.5, (T,)), 1, -1)
    idx = jnp.arange(T)
    direction = jnp.where(idx == 0, 1, direction)
    direction = jnp.where(idx == T - 1, -1, direction)
    return jnp.where(replace, idx + direction, idx)


def _jitter_body(x_ref, d_ref, o_ref):
    x = x_ref[...]
    d = d_ref[...]
    left = jnp.concatenate([x[:, :1], x[:, :-1]], axis=1)
    right = jnp.concatenate([x[:, 1:], x[:, -1:]], axis=1)
    o_ref[...] = jnp.where(d == -1, left, jnp.where(d == 1, right, x))


def kernel(quantized):
    B, C, T = quantized.shape
    x = quantized.reshape(B * C, T)
    R = B * C
    BLK = 256
    neighbor = _neighbor_indices(T)
    delta = (neighbor - jnp.arange(T)).astype(jnp.int32).reshape(1, T)

    out = pl.pallas_call(
        _jitter_body,
        grid=(R // BLK,),
        in_specs=[
            pl.BlockSpec((BLK, T), lambda i: (i, 0)),
            pl.BlockSpec((1, T), lambda i: (0, 0)),
        ],
        out_specs=pl.BlockSpec((BLK, T), lambda i: (i, 0)),
        out_shape=jax.ShapeDtypeStruct((R, T), jnp.float32),
    )(x, delta)
    return out.reshape(B, C, T)
